# baseline (speedup 1.0000x reference)
import functools

import jax
import jax.numpy as jnp
from jax import lax
from jax.experimental import pallas as pl
from jax.experimental.pallas import tpu as pltpu
from jax.experimental.pallas import tpu_sc as plsc

N_TASKS = 10000
FRONTIER = 10
X_DIM = 2048
CODE_DIM = 512
BATCH = 4096

NC, NS, L = 2, 16, 16
NW = NC * NS
CHUNK = BATCH // NW
NGROUPS = CHUNK // L
NVEC = BATCH // L
SW = 128

BM = 512
GRID = BATCH // BM


def _enc_body(x_ref, we_ref, wd_ref, score_ref, c_ref):
    xb = x_ref[...].astype(jnp.bfloat16)
    c = jnp.tanh(jnp.dot(xb, we_ref[...].astype(jnp.bfloat16),
                         preferred_element_type=jnp.float32))
    xr = jnp.dot(c.astype(jnp.bfloat16), wd_ref[...].astype(jnp.bfloat16),
                 preferred_element_type=jnp.float32)
    d = x_ref[...] - xr
    score = -0.5 * (jnp.sum(d * d, axis=1) + jnp.sum(c * c, axis=1))
    score_ref[...] = score.reshape(1, 1, BM)
    c_ref[...] = c


def _dec_body(x_ref, c_ref, wd_ref, se_ref, obj_ref):
    c = c_ref[...]
    xr = jnp.dot(c.astype(jnp.bfloat16), wd_ref[...].astype(jnp.bfloat16),
                 preferred_element_type=jnp.float32)
    d = x_ref[...] - xr
    lik = -0.5 * jnp.sum(d * d, axis=1)
    prior = -0.5 * jnp.sum(c * c, axis=1)
    obj = lik + jnp.log(se_ref[0, 0, :]) + prior
    obj_ref[...] = obj.reshape(1, 1, BM)


def _sc_body(i_hbm, score_hbm, mem_s_hbm, mem_c_hbm, c_prop_hbm,
             cs_hbm, sumexp_hbm,
             i_loc, score_loc, wlast,
             i_chunk, srows, sumexp_chunk, j_vec, w_vec, dst_vec, cbuf, sem):
    cid = lax.axis_index("c")
    sid = lax.axis_index("s")
    wid = cid * NS + sid
    base = wid * CHUNK
    lane = jnp.arange(L, dtype=jnp.int32)

    pltpu.sync_copy(i_hbm, i_loc)
    pltpu.sync_copy(score_hbm, score_loc)

    def table_step(v, carry):
        idx16 = i_loc[pl.ds(v * L, L)]
        b16 = lane + v * L
        for l in range(L):
            idx_l = jnp.where(lane == l, idx16, N_TASKS)
            plsc.store_scatter(wlast, [idx_l], b16)
        return carry

    lax.fori_loop(0, NVEC, table_step, 0)

    pltpu.sync_copy(i_hbm.at[pl.ds(base, CHUNK)], i_chunk)
    pltpu.async_copy(mem_s_hbm.at[i_chunk], srows, sem).wait()

    for g in range(NGROUPS):
        iv = i_chunk[pl.ds(g * L, L)]
        row0 = lane + g * L
        sv = [plsc.load_gather(srows, [row0, jnp.full((L,), f, jnp.int32)])
              for f in range(FRONTIER)]
        cur_min = sv[0]
        for f in range(1, FRONTIER):
            cur_min = jnp.minimum(cur_min, sv[f])
        m = jnp.full((L,), 99, dtype=jnp.int32)
        for f in range(FRONTIER):
            m = jnp.minimum(m, jnp.where(sv[f] == cur_min, f, 99))
        w = plsc.load_gather(wlast, [iv])
        score_w = plsc.load_gather(score_loc, [w])
        better = score_w > cur_min
        new_s = jnp.where(better, score_w, cur_min)
        upd = [jnp.where(m == f, new_s, sv[f]) for f in range(FRONTIER)]
        vmax = upd[0]
        for f in range(1, FRONTIER):
            vmax = jnp.maximum(vmax, upd[f])
        j = jnp.full((L,), 99, dtype=jnp.int32)
        for f in range(FRONTIER):
            j = jnp.minimum(j, jnp.where(upd[f] == vmax, f, 99))
        sumexp = jnp.exp(upd[0] - vmax)
        for f in range(1, FRONTIER):
            sumexp = sumexp + jnp.exp(upd[f] - vmax)
        use_new = (j == m) & better
        bglob = base + g * L + lane
        sumexp_chunk[pl.ds(g * L, L)] = sumexp
        j_vec[pl.ds(g * L, L)] = j * N_TASKS + iv
        w_vec[pl.ds(g * L, L)] = w
        dst_vec[pl.ds(g * L, L)] = jnp.where(use_new, bglob, BATCH)
    pltpu.sync_copy(sumexp_chunk, sumexp_hbm.at[pl.ds(base, CHUNK)])

    pltpu.async_copy(mem_c_hbm.at[j_vec], cbuf, sem).wait()
    pltpu.sync_copy(cbuf, cs_hbm.at[pl.ds(base, CHUNK)])
    pltpu.async_copy(c_prop_hbm.at[w_vec], cbuf, sem).wait()
    pltpu.async_copy(cbuf, cs_hbm.at[dst_vec], sem).wait()


_sc_call = pl.kernel(
    _sc_body,
    out_type=[
        jax.ShapeDtypeStruct((BATCH + 8, CODE_DIM), jnp.float32),
        jax.ShapeDtypeStruct((BATCH,), jnp.float32),
    ],
    mesh=plsc.VectorSubcoreMesh(core_axis_name="c", subcore_axis_name="s"),
    compiler_params=pltpu.CompilerParams(needs_layout_passes=False),
    scratch_types=[
        pltpu.VMEM((BATCH,), jnp.int32),
        pltpu.VMEM((BATCH,), jnp.float32),
        pltpu.VMEM((N_TASKS + L,), jnp.int32),
        pltpu.VMEM((CHUNK,), jnp.int32),
        pltpu.VMEM((CHUNK, SW), jnp.float32),
        pltpu.VMEM((CHUNK,), jnp.float32),
        pltpu.VMEM((CHUNK,), jnp.int32),
        pltpu.VMEM((CHUNK,), jnp.int32),
        pltpu.VMEM((CHUNK,), jnp.int32),
        pltpu.VMEM((CHUNK, CODE_DIM), jnp.float32),
        pltpu.SemaphoreType.DMA,
    ],
)


def kernel(x, i, W_enc, W_dec, mem_c, mem_s):
    score3, c_prop = pl.pallas_call(
        _enc_body,
        grid=(GRID,),
        in_specs=[
            pl.BlockSpec((BM, X_DIM), lambda b: (b, 0)),
            pl.BlockSpec((X_DIM, CODE_DIM), lambda b: (0, 0)),
            pl.BlockSpec((CODE_DIM, X_DIM), lambda b: (0, 0)),
        ],
        out_specs=[
            pl.BlockSpec((1, 1, BM), lambda b: (b, 0, 0)),
            pl.BlockSpec((BM, CODE_DIM), lambda b: (b, 0)),
        ],
        out_shape=[
            jax.ShapeDtypeStruct((GRID, 1, BM), jnp.float32),
            jax.ShapeDtypeStruct((BATCH, CODE_DIM), jnp.float32),
        ],
    )(x, W_enc, W_dec)
    score = score3.reshape(BATCH)

    mem_s_pad = jnp.pad(mem_s, ((0, 0), (0, SW - FRONTIER)))
    mem_c_flat = jnp.transpose(mem_c, (1, 0, 2)).reshape(
        FRONTIER * N_TASKS, CODE_DIM)
    c_sel_ext, sumexp = _sc_call(i, score, mem_s_pad, mem_c_flat, c_prop)

    obj3 = pl.pallas_call(
        _dec_body,
        grid=(GRID,),
        in_specs=[
            pl.BlockSpec((BM, X_DIM), lambda b: (b, 0)),
            pl.BlockSpec((BM, CODE_DIM), lambda b: (b, 0)),
            pl.BlockSpec((CODE_DIM, X_DIM), lambda b: (0, 0)),
            pl.BlockSpec((1, 1, BM), lambda b: (b, 0, 0)),
        ],
        out_specs=pl.BlockSpec((1, 1, BM), lambda b: (b, 0, 0)),
        out_shape=jax.ShapeDtypeStruct((GRID, 1, BM), jnp.float32),
    )(x, c_sel_ext, W_dec, sumexp.reshape(GRID, 1, BM))
    return obj3.reshape(BATCH)


# speedup vs baseline: 1.0435x; 1.0435x over previous
import functools

import jax
import jax.numpy as jnp
from jax import lax
from jax.experimental import pallas as pl
from jax.experimental.pallas import tpu as pltpu
from jax.experimental.pallas import tpu_sc as plsc

N_TASKS = 10000
FRONTIER = 10
X_DIM = 2048
CODE_DIM = 512
BATCH = 4096

NC, NS, L = 2, 16, 16
NW = NC * NS
CHUNK = BATCH // NW
NGROUPS = CHUNK // L
NVEC = BATCH // L
SW = 128

BM = 512
GRID = BATCH // BM


def _enc_body(x_ref, we_ref, wd_ref, score_ref, c_ref):
    c = jnp.tanh(jnp.dot(x_ref[...], we_ref[...],
                         preferred_element_type=jnp.float32))
    xr = jnp.dot(c, wd_ref[...], preferred_element_type=jnp.float32)
    d = x_ref[...] - xr
    score = -0.5 * (jnp.sum(d * d, axis=1) + jnp.sum(c * c, axis=1))
    score_ref[...] = score.reshape(1, 1, BM)
    c_ref[...] = c


def _dec_body(x_ref, c_ref, wd_ref, se_ref, obj_ref):
    c = c_ref[...]
    xr = jnp.dot(c, wd_ref[...], preferred_element_type=jnp.float32)
    d = x_ref[...] - xr
    lik = -0.5 * jnp.sum(d * d, axis=1)
    prior = -0.5 * jnp.sum(c * c, axis=1)
    obj = lik + jnp.log(se_ref[0, 0, :]) + prior
    obj_ref[...] = obj.reshape(1, 1, BM)


def _sc_table_body(i_hbm, w_hbm, i_loc, wlast, w_chunk, sem):
    cid = lax.axis_index("c")
    sid = lax.axis_index("s")
    wid = cid * NS + sid
    base = wid * CHUNK
    lane = jnp.arange(L, dtype=jnp.int32)

    pltpu.sync_copy(i_hbm, i_loc)

    def table_step(v, carry):
        idx16 = i_loc[pl.ds(v * L, L)]
        b16 = lane + v * L
        for l in range(L):
            idx_l = jnp.where(lane == l, idx16, N_TASKS)
            plsc.store_scatter(wlast, [idx_l], b16)
        return carry

    lax.fori_loop(0, NVEC, table_step, 0)

    for g in range(NGROUPS):
        iv = i_loc[pl.ds(base + g * L, L)]
        w_chunk[pl.ds(g * L, L)] = plsc.load_gather(wlast, [iv])
    pltpu.sync_copy(w_chunk, w_hbm.at[pl.ds(base, CHUNK)])


def _sc_body(i_hbm, score_hbm, mem_s_hbm, mem_c_hbm, c_prop_hbm, w_hbm,
             cs_hbm, sumexp_hbm,
             score_loc, w_loc,
             i_chunk, srows, sumexp_chunk, j_vec, w_vec, dst_vec, cbuf, sem):
    cid = lax.axis_index("c")
    sid = lax.axis_index("s")
    wid = cid * NS + sid
    base = wid * CHUNK
    lane = jnp.arange(L, dtype=jnp.int32)

    pltpu.sync_copy(score_hbm, score_loc)
    pltpu.sync_copy(w_hbm.at[pl.ds(base, CHUNK)], w_loc)

    pltpu.sync_copy(i_hbm.at[pl.ds(base, CHUNK)], i_chunk)
    pltpu.async_copy(mem_s_hbm.at[i_chunk], srows, sem).wait()

    for g in range(NGROUPS):
        iv = i_chunk[pl.ds(g * L, L)]
        row0 = lane + g * L
        sv = [plsc.load_gather(srows, [row0, jnp.full((L,), f, jnp.int32)])
              for f in range(FRONTIER)]
        cur_min = sv[0]
        for f in range(1, FRONTIER):
            cur_min = jnp.minimum(cur_min, sv[f])
        m = jnp.full((L,), 99, dtype=jnp.int32)
        for f in range(FRONTIER):
            m = jnp.minimum(m, jnp.where(sv[f] == cur_min, f, 99))
        w = w_loc[pl.ds(g * L, L)]
        score_w = plsc.load_gather(score_loc, [w])
        better = score_w > cur_min
        new_s = jnp.where(better, score_w, cur_min)
        upd = [jnp.where(m == f, new_s, sv[f]) for f in range(FRONTIER)]
        vmax = upd[0]
        for f in range(1, FRONTIER):
            vmax = jnp.maximum(vmax, upd[f])
        j = jnp.full((L,), 99, dtype=jnp.int32)
        for f in range(FRONTIER):
            j = jnp.minimum(j, jnp.where(upd[f] == vmax, f, 99))
        sumexp = jnp.exp(upd[0] - vmax)
        for f in range(1, FRONTIER):
            sumexp = sumexp + jnp.exp(upd[f] - vmax)
        use_new = (j == m) & better
        bglob = base + g * L + lane
        sumexp_chunk[pl.ds(g * L, L)] = sumexp
        j_vec[pl.ds(g * L, L)] = j * N_TASKS + iv
        w_vec[pl.ds(g * L, L)] = w
        dst_vec[pl.ds(g * L, L)] = jnp.where(use_new, bglob, BATCH)
    pltpu.sync_copy(sumexp_chunk, sumexp_hbm.at[pl.ds(base, CHUNK)])

    pltpu.async_copy(mem_c_hbm.at[j_vec], cbuf, sem).wait()
    pltpu.sync_copy(cbuf, cs_hbm.at[pl.ds(base, CHUNK)])
    pltpu.async_copy(c_prop_hbm.at[w_vec], cbuf, sem).wait()
    pltpu.async_copy(cbuf, cs_hbm.at[dst_vec], sem).wait()


_sc_table_call = pl.kernel(
    _sc_table_body,
    out_type=[jax.ShapeDtypeStruct((BATCH,), jnp.int32)],
    mesh=plsc.VectorSubcoreMesh(core_axis_name="c", subcore_axis_name="s"),
    compiler_params=pltpu.CompilerParams(needs_layout_passes=False),
    scratch_types=[
        pltpu.VMEM((BATCH,), jnp.int32),
        pltpu.VMEM((N_TASKS + L,), jnp.int32),
        pltpu.VMEM((CHUNK,), jnp.int32),
        pltpu.SemaphoreType.DMA,
    ],
)

_sc_call = pl.kernel(
    _sc_body,
    out_type=[
        jax.ShapeDtypeStruct((BATCH + 8, CODE_DIM), jnp.float32),
        jax.ShapeDtypeStruct((BATCH,), jnp.float32),
    ],
    mesh=plsc.VectorSubcoreMesh(core_axis_name="c", subcore_axis_name="s"),
    compiler_params=pltpu.CompilerParams(needs_layout_passes=False),
    scratch_types=[
        pltpu.VMEM((BATCH,), jnp.float32),
        pltpu.VMEM((CHUNK,), jnp.int32),
        pltpu.VMEM((CHUNK,), jnp.int32),
        pltpu.VMEM((CHUNK, SW), jnp.float32),
        pltpu.VMEM((CHUNK,), jnp.float32),
        pltpu.VMEM((CHUNK,), jnp.int32),
        pltpu.VMEM((CHUNK,), jnp.int32),
        pltpu.VMEM((CHUNK,), jnp.int32),
        pltpu.VMEM((CHUNK, CODE_DIM), jnp.float32),
        pltpu.SemaphoreType.DMA,
    ],
)


def kernel(x, i, W_enc, W_dec, mem_c, mem_s):
    score3, c_prop = pl.pallas_call(
        _enc_body,
        grid=(GRID,),
        in_specs=[
            pl.BlockSpec((BM, X_DIM), lambda b: (b, 0)),
            pl.BlockSpec((X_DIM, CODE_DIM), lambda b: (0, 0)),
            pl.BlockSpec((CODE_DIM, X_DIM), lambda b: (0, 0)),
        ],
        out_specs=[
            pl.BlockSpec((1, 1, BM), lambda b: (b, 0, 0)),
            pl.BlockSpec((BM, CODE_DIM), lambda b: (b, 0)),
        ],
        out_shape=[
            jax.ShapeDtypeStruct((GRID, 1, BM), jnp.float32),
            jax.ShapeDtypeStruct((BATCH, CODE_DIM), jnp.float32),
        ],
    )(x, W_enc, W_dec)
    score = score3.reshape(BATCH)

    mem_s_pad = jnp.pad(mem_s, ((0, 0), (0, SW - FRONTIER)))
    mem_c_flat = jnp.transpose(mem_c, (1, 0, 2)).reshape(
        FRONTIER * N_TASKS, CODE_DIM)
    (wfull,) = _sc_table_call(i)
    c_sel_ext, sumexp = _sc_call(
        i, score, mem_s_pad, mem_c_flat, c_prop, wfull)

    obj3 = pl.pallas_call(
        _dec_body,
        grid=(GRID,),
        in_specs=[
            pl.BlockSpec((BM, X_DIM), lambda b: (b, 0)),
            pl.BlockSpec((BM, CODE_DIM), lambda b: (b, 0)),
            pl.BlockSpec((CODE_DIM, X_DIM), lambda b: (0, 0)),
            pl.BlockSpec((1, 1, BM), lambda b: (b, 0, 0)),
        ],
        out_specs=pl.BlockSpec((1, 1, BM), lambda b: (b, 0, 0)),
        out_shape=jax.ShapeDtypeStruct((GRID, 1, BM), jnp.float32),
    )(x, c_sel_ext, W_dec, sumexp.reshape(GRID, 1, BM))
    return obj3.reshape(BATCH)


# speedup vs baseline: 1.1233x; 1.0765x over previous
import functools

import jax
import jax.numpy as jnp
from jax import lax
from jax.experimental import pallas as pl
from jax.experimental.pallas import tpu as pltpu
from jax.experimental.pallas import tpu_sc as plsc

N_TASKS = 10000
FRONTIER = 10
X_DIM = 2048
CODE_DIM = 512
BATCH = 4096

NC, NS, L = 2, 16, 16
NW = NC * NS
CHUNK = BATCH // NW
NGROUPS = CHUNK // L
NVEC = BATCH // L
SW = 128

BM = 512
GRID = BATCH // BM


def _enc_body(x_ref, we_ref, wd_ref, score_ref, c_ref):
    c = jnp.tanh(jnp.dot(x_ref[...], we_ref[...],
                         preferred_element_type=jnp.float32))
    xr = jnp.dot(c, wd_ref[...], preferred_element_type=jnp.float32)
    d = x_ref[...] - xr
    score = -0.5 * (jnp.sum(d * d, axis=1) + jnp.sum(c * c, axis=1))
    score_ref[...] = score.reshape(1, 1, BM)
    c_ref[...] = c


def _dec_body(x_ref, c_ref, wd_ref, se_ref, obj_ref):
    c = c_ref[...]
    xr = jnp.dot(c, wd_ref[...], preferred_element_type=jnp.float32)
    d = x_ref[...] - xr
    lik = -0.5 * jnp.sum(d * d, axis=1)
    prior = -0.5 * jnp.sum(c * c, axis=1)
    obj = lik + jnp.log(se_ref[0, 0, :]) + prior
    obj_ref[...] = obj.reshape(1, 1, BM)


def _sc_table_body(i_hbm, w_hbm, i_loc, wlast, w_chunk, sem):
    cid = lax.axis_index("c")
    sid = lax.axis_index("s")
    wid = cid * NS + sid
    base = wid * CHUNK
    lane = jnp.arange(L, dtype=jnp.int32)

    pltpu.sync_copy(i_hbm, i_loc)

    def table_step(v, carry):
        idx16 = i_loc[pl.ds(v * L, L)]
        b16 = lane + v * L
        for l in range(L):
            idx_l = jnp.where(lane == l, idx16, N_TASKS)
            plsc.store_scatter(wlast, [idx_l], b16)
        return carry

    lax.fori_loop(0, NVEC, table_step, 0)

    for g in range(NGROUPS):
        iv = i_loc[pl.ds(base + g * L, L)]
        w_chunk[pl.ds(g * L, L)] = plsc.load_gather(wlast, [iv])
    pltpu.sync_copy(w_chunk, w_hbm.at[pl.ds(base, CHUNK)])


def _sc_body(i_hbm, score_hbm, mem_s_hbm, mem_c_hbm, c_prop_hbm, w_hbm,
             cs_hbm, sumexp_hbm,
             score_loc, w_loc,
             i_chunk, srows, sumexp_chunk, j_vec, w_vec, dst_vec, cbuf, sem):
    cid = lax.axis_index("c")
    sid = lax.axis_index("s")
    wid = cid * NS + sid
    base = wid * CHUNK
    lane = jnp.arange(L, dtype=jnp.int32)

    pltpu.sync_copy(score_hbm, score_loc)
    pltpu.sync_copy(w_hbm.at[pl.ds(base, CHUNK)], w_loc)

    pltpu.sync_copy(i_hbm.at[pl.ds(base, CHUNK)], i_chunk)
    pltpu.async_copy(mem_s_hbm.at[i_chunk], srows, sem).wait()

    n_mem = jnp.int32(0)
    for g in range(NGROUPS):
        iv = i_chunk[pl.ds(g * L, L)]
        row0 = lane + g * L
        sv = [plsc.load_gather(srows, [row0, jnp.full((L,), f, jnp.int32)])
              for f in range(FRONTIER)]
        cur_min = sv[0]
        for f in range(1, FRONTIER):
            cur_min = jnp.minimum(cur_min, sv[f])
        m = jnp.full((L,), 99, dtype=jnp.int32)
        for f in range(FRONTIER):
            m = jnp.minimum(m, jnp.where(sv[f] == cur_min, f, 99))
        w = w_loc[pl.ds(g * L, L)]
        score_w = plsc.load_gather(score_loc, [w])
        better = score_w > cur_min
        new_s = jnp.where(better, score_w, cur_min)
        upd = [jnp.where(m == f, new_s, sv[f]) for f in range(FRONTIER)]
        vmax = upd[0]
        for f in range(1, FRONTIER):
            vmax = jnp.maximum(vmax, upd[f])
        j = jnp.full((L,), 99, dtype=jnp.int32)
        for f in range(FRONTIER):
            j = jnp.minimum(j, jnp.where(upd[f] == vmax, f, 99))
        sumexp = jnp.exp(upd[0] - vmax)
        for f in range(1, FRONTIER):
            sumexp = sumexp + jnp.exp(upd[f] - vmax)
        use_new = (j == m) & better
        bglob = base + g * L + lane
        sumexp_chunk[pl.ds(g * L, L)] = sumexp
        j_vec[pl.ds(g * L, L)] = j * N_TASKS + iv
        w_vec[pl.ds(g * L, L)] = w
        dst_vec[pl.ds(g * L, L)] = jnp.where(use_new, BATCH, bglob)
        n_mem = n_mem + jnp.sum(jnp.where(use_new, 0, 1))
    pltpu.sync_copy(sumexp_chunk, sumexp_hbm.at[pl.ds(base, CHUNK)])

    pltpu.async_copy(c_prop_hbm.at[w_vec], cbuf, sem).wait()
    pltpu.sync_copy(cbuf, cs_hbm.at[pl.ds(base, CHUNK)])

    @pl.when(n_mem > 0)
    def _patch():
        pltpu.async_copy(mem_c_hbm.at[j_vec], cbuf, sem).wait()
        pltpu.async_copy(cbuf, cs_hbm.at[dst_vec], sem).wait()


_sc_table_call = pl.kernel(
    _sc_table_body,
    out_type=[jax.ShapeDtypeStruct((BATCH,), jnp.int32)],
    mesh=plsc.VectorSubcoreMesh(core_axis_name="c", subcore_axis_name="s"),
    compiler_params=pltpu.CompilerParams(needs_layout_passes=False),
    scratch_types=[
        pltpu.VMEM((BATCH,), jnp.int32),
        pltpu.VMEM((N_TASKS + L,), jnp.int32),
        pltpu.VMEM((CHUNK,), jnp.int32),
        pltpu.SemaphoreType.DMA,
    ],
)

_sc_call = pl.kernel(
    _sc_body,
    out_type=[
        jax.ShapeDtypeStruct((BATCH + 8, CODE_DIM), jnp.float32),
        jax.ShapeDtypeStruct((BATCH,), jnp.float32),
    ],
    mesh=plsc.VectorSubcoreMesh(core_axis_name="c", subcore_axis_name="s"),
    compiler_params=pltpu.CompilerParams(needs_layout_passes=False),
    scratch_types=[
        pltpu.VMEM((BATCH,), jnp.float32),
        pltpu.VMEM((CHUNK,), jnp.int32),
        pltpu.VMEM((CHUNK,), jnp.int32),
        pltpu.VMEM((CHUNK, SW), jnp.float32),
        pltpu.VMEM((CHUNK,), jnp.float32),
        pltpu.VMEM((CHUNK,), jnp.int32),
        pltpu.VMEM((CHUNK,), jnp.int32),
        pltpu.VMEM((CHUNK,), jnp.int32),
        pltpu.VMEM((CHUNK, CODE_DIM), jnp.float32),
        pltpu.SemaphoreType.DMA,
    ],
)


def kernel(x, i, W_enc, W_dec, mem_c, mem_s):
    score3, c_prop = pl.pallas_call(
        _enc_body,
        grid=(GRID,),
        in_specs=[
            pl.BlockSpec((BM, X_DIM), lambda b: (b, 0)),
            pl.BlockSpec((X_DIM, CODE_DIM), lambda b: (0, 0)),
            pl.BlockSpec((CODE_DIM, X_DIM), lambda b: (0, 0)),
        ],
        out_specs=[
            pl.BlockSpec((1, 1, BM), lambda b: (b, 0, 0)),
            pl.BlockSpec((BM, CODE_DIM), lambda b: (b, 0)),
        ],
        out_shape=[
            jax.ShapeDtypeStruct((GRID, 1, BM), jnp.float32),
            jax.ShapeDtypeStruct((BATCH, CODE_DIM), jnp.float32),
        ],
    )(x, W_enc, W_dec)
    score = score3.reshape(BATCH)

    mem_s_pad = jnp.pad(mem_s, ((0, 0), (0, SW - FRONTIER)))
    mem_c_flat = jnp.transpose(mem_c, (1, 0, 2)).reshape(
        FRONTIER * N_TASKS, CODE_DIM)
    (wfull,) = _sc_table_call(i)
    c_sel_ext, sumexp = _sc_call(
        i, score, mem_s_pad, mem_c_flat, c_prop, wfull)

    obj3 = pl.pallas_call(
        _dec_body,
        grid=(GRID,),
        in_specs=[
            pl.BlockSpec((BM, X_DIM), lambda b: (b, 0)),
            pl.BlockSpec((BM, CODE_DIM), lambda b: (b, 0)),
            pl.BlockSpec((CODE_DIM, X_DIM), lambda b: (0, 0)),
            pl.BlockSpec((1, 1, BM), lambda b: (b, 0, 0)),
        ],
        out_specs=pl.BlockSpec((1, 1, BM), lambda b: (b, 0, 0)),
        out_shape=jax.ShapeDtypeStruct((GRID, 1, BM), jnp.float32),
    )(x, c_sel_ext, W_dec, sumexp.reshape(GRID, 1, BM))
    return obj3.reshape(BATCH)
